# Initial kernel scaffold; baseline (speedup 1.0000x reference)
import jax
import jax.numpy as jnp
from jax.experimental import pallas as pl


def kernel(x, atom_type, aa_type):
    raise NotImplementedError("write your pallas kernel here")


# speedup vs baseline: 1.6193x; 1.6193x over previous
import functools

import jax
import jax.numpy as jnp
from jax import lax
from jax.experimental import pallas as pl
from jax.experimental.pallas import tpu as pltpu
from jax.experimental.pallas import tpu_sc as plsc

B, L, D = 16, 4096, 128
NC, NS = 2, 16
NW = NC * NS
HALF = L // 2
LANES = 16
NG = HALF // LANES
CHUNK = 128
NCH = HALF // CHUNK


def _sc_body(x_hbm, atom_hbm, out_hbm, atom_v, idx_v, xbuf, sem):
    c = lax.axis_index("c")
    s = lax.axis_index("s")
    wid = c * NS + s
    b = wid // 2
    h = wid % 2
    row_base = b * L
    my_base = row_base + h * HALF

    pltpu.sync_copy(atom_hbm.at[pl.ds(row_base, L)], atom_v)

    zeros = jnp.zeros((LANES,), jnp.int32)

    def count_half(half):
        def grp(g, acc):
            a = atom_v[pl.ds(half * HALF + g * LANES, LANES)]
            return acc + jnp.where(a < 3, 1, 0).astype(jnp.int32)
        acc = lax.fori_loop(0, NG, grp, zeros)
        return jnp.sum(acc)

    c0 = count_half(0)
    c1 = count_half(1)

    bb_base = row_base + h * c0
    ot_base = row_base + (c0 + c1) + h * (HALF - c0)
    lane = lax.iota(jnp.int32, LANES)

    for j in range(NCH):
        def grp(g, run):
            a = atom_v[pl.ds(h * HALF + j * CHUNK + g * LANES, LANES)]
            flag = a < 3
            f = jnp.where(flag, 1, 0).astype(jnp.int32)
            cs = plsc.cumsum(f)
            p = lane + (j * CHUNK + g * LANES)
            ib = run + cs
            dest = jnp.where(flag, bb_base + ib - 1, ot_base + p - ib)
            idx_v[j, pl.ds(g * LANES, LANES)] = dest
            return run + jnp.sum(f)
        run0 = jnp.int32(0) if j == 0 else run
        run = lax.fori_loop(0, CHUNK // LANES, grp, run0)

    for j in range(NCH):
        pltpu.sync_copy(x_hbm.at[pl.ds(my_base + j * CHUNK, CHUNK)], xbuf)
        pltpu.async_copy(xbuf, out_hbm.at[idx_v.at[j]], sem).wait()


@jax.jit
def kernel(x, atom_type, aa_type):
    del aa_type
    x2 = x.reshape(B * L, D)
    at = atom_type.astype(jnp.int32).reshape(B * L)
    run = pl.kernel(
        _sc_body,
        out_type=jax.ShapeDtypeStruct((B * L, D), jnp.float32),
        mesh=plsc.VectorSubcoreMesh(core_axis_name="c", subcore_axis_name="s"),
        compiler_params=pltpu.CompilerParams(needs_layout_passes=False),
        scratch_types=[
            pltpu.VMEM((L,), jnp.int32),
            pltpu.VMEM((NCH, CHUNK), jnp.int32),
            pltpu.VMEM((CHUNK, D), jnp.float32),
            pltpu.SemaphoreType.DMA,
        ],
    )(x2, at)
    return run.reshape(B, L, D)


# speedup vs baseline: 2.0629x; 1.2739x over previous
import functools

import jax
import jax.numpy as jnp
from jax import lax
from jax.experimental import pallas as pl
from jax.experimental.pallas import tpu as pltpu
from jax.experimental.pallas import tpu_sc as plsc

B, L, D = 16, 4096, 128
NC, NS = 2, 16
NW = NC * NS
HALF = L // 2
LANES = 16
NG = HALF // LANES
CHUNK = 128
NCH = HALF // CHUNK


NBUF = 4


def _sc_body(x_hbm, atom_hbm, out_hbm, atom_v, idx_v, xbuf, gsems, ssems):
    c = lax.axis_index("c")
    s = lax.axis_index("s")
    wid = c * NS + s
    b = wid // 2
    h = wid % 2
    row_base = b * L
    my_base = row_base + h * HALF

    gat = [None] * NBUF
    scat = [None] * NBUF
    for j in range(2):
        gat[j] = pltpu.async_copy(
            x_hbm.at[pl.ds(my_base + j * CHUNK, CHUNK)], xbuf.at[j], gsems.at[j])

    pltpu.sync_copy(atom_hbm.at[pl.ds(row_base, L)], atom_v)

    zeros = jnp.zeros((LANES,), jnp.int32)

    def count_half(half):
        def grp(g, acc):
            a = atom_v[pl.ds(half * HALF + g * LANES, LANES)]
            return acc + jnp.where(a < 3, 1, 0).astype(jnp.int32)
        acc = lax.fori_loop(0, NG, grp, zeros)
        return jnp.sum(acc)

    c0 = count_half(0)
    c1 = count_half(1)

    bb_base = row_base + h * c0
    ot_base = row_base + (c0 + c1) + h * (HALF - c0)
    lane = lax.iota(jnp.int32, LANES)

    for j in range(NCH):
        def grp(g, run):
            a = atom_v[pl.ds(h * HALF + j * CHUNK + g * LANES, LANES)]
            flag = a < 3
            f = jnp.where(flag, 1, 0).astype(jnp.int32)
            cs = plsc.cumsum(f)
            p = lane + (j * CHUNK + g * LANES)
            ib = run + cs
            dest = jnp.where(flag, bb_base + ib - 1, ot_base + p - ib)
            idx_v[j, pl.ds(g * LANES, LANES)] = dest
            return run + jnp.sum(f)
        run0 = jnp.int32(0) if j == 0 else run
        run = lax.fori_loop(0, CHUNK // LANES, grp, run0)

    for j in range(NCH):
        jn = j + 2
        if jn < NCH:
            bn = jn % NBUF
            if scat[bn] is not None:
                scat[bn].wait()
            gat[bn] = pltpu.async_copy(
                x_hbm.at[pl.ds(my_base + jn * CHUNK, CHUNK)], xbuf.at[bn],
                gsems.at[bn])
        bj = j % NBUF
        gat[bj].wait()
        scat[bj] = pltpu.async_copy(xbuf.at[bj], out_hbm.at[idx_v.at[j]],
                                    ssems.at[bj])
    for bj in range(NBUF):
        scat[bj].wait()


@jax.jit
def kernel(x, atom_type, aa_type):
    del aa_type
    x2 = x.reshape(B * L, D)
    at = atom_type.astype(jnp.int32).reshape(B * L)
    run = pl.kernel(
        _sc_body,
        out_type=jax.ShapeDtypeStruct((B * L, D), jnp.float32),
        mesh=plsc.VectorSubcoreMesh(core_axis_name="c", subcore_axis_name="s"),
        compiler_params=pltpu.CompilerParams(needs_layout_passes=False),
        scratch_types=[
            pltpu.VMEM((L,), jnp.int32),
            pltpu.VMEM((NCH, CHUNK), jnp.int32),
            pltpu.VMEM((NBUF, CHUNK, D), jnp.float32),
            pltpu.SemaphoreType.DMA((NBUF,)),
            pltpu.SemaphoreType.DMA((NBUF,)),
        ],
    )(x2, at)
    return run.reshape(B, L, D)


# speedup vs baseline: 2.1599x; 1.0470x over previous
import jax
import jax.numpy as jnp
from jax import lax
from jax.experimental import pallas as pl
from jax.experimental.pallas import tpu as pltpu
from jax.experimental.pallas import tpu_sc as plsc

B, L, D = 16, 4096, 128
NC, NS = 2, 16
HALF = L // 2
LANES = 16
NG = HALF // LANES
CHUNK = 128
NCH = HALF // CHUNK
GPC = CHUNK // LANES
NBUF = 4


def _sc_body(x_hbm, atom_hbm, out_hbm, atom_v, idx_v, xbuf, gsems, ssems):
    c = lax.axis_index("c")
    s = lax.axis_index("s")
    wid = c * NS + s
    b = wid // 2
    h = wid % 2
    row_base = b * L
    my_base = row_base + h * HALF

    gat = [None] * NBUF
    scat = [None] * NBUF
    for j in range(2):
        gat[j] = pltpu.async_copy(
            x_hbm.at[pl.ds(my_base + j * CHUNK, CHUNK)], xbuf.at[j], gsems.at[j])

    pltpu.sync_copy(atom_hbm.at[pl.ds(row_base, L)], atom_v)

    def count_half(half):
        def grp(g, acc):
            accs = []
            for u in range(4):
                a = atom_v[pl.ds(half * HALF + (4 * g + u) * LANES, LANES)]
                accs.append(jnp.where(a < 3, 1, 0).astype(jnp.int32))
            return acc + ((accs[0] + accs[1]) + (accs[2] + accs[3]))
        acc = lax.fori_loop(0, NG // 4, grp, jnp.zeros((LANES,), jnp.int32))
        return jnp.sum(acc)

    c0 = count_half(0)
    c1 = count_half(1)

    bb_base = row_base + h * c0
    ot_base = row_base + (c0 + c1) + h * (HALF - c0)
    lane = lax.iota(jnp.int32, LANES)

    run = jnp.int32(0)
    for j in range(NCH):
        def grp(g, run):
            a = atom_v[pl.ds(h * HALF + j * CHUNK + g * LANES, LANES)]
            flag = a < 3
            f = jnp.where(flag, 1, 0).astype(jnp.int32)
            cs = plsc.cumsum(f)
            p = lane + (j * CHUNK + g * LANES)
            ib = run + cs
            dest = jnp.where(flag, bb_base + ib - 1, ot_base + p - ib)
            idx_v[j, pl.ds(g * LANES, LANES)] = dest
            return run + jnp.sum(f)
        run = lax.fori_loop(0, GPC, grp, run)

        jn = j + 2
        if jn < NCH:
            bn = jn % NBUF
            if scat[bn] is not None:
                scat[bn].wait()
            gat[bn] = pltpu.async_copy(
                x_hbm.at[pl.ds(my_base + jn * CHUNK, CHUNK)], xbuf.at[bn],
                gsems.at[bn])
        bj = j % NBUF
        gat[bj].wait()
        scat[bj] = pltpu.async_copy(xbuf.at[bj], out_hbm.at[idx_v.at[j]],
                                    ssems.at[bj])

    for bj in range(NBUF):
        scat[bj].wait()


@jax.jit
def kernel(x, atom_type, aa_type):
    del aa_type
    x2 = x.reshape(B * L, D)
    at = atom_type.astype(jnp.int32).reshape(B * L)
    run = pl.kernel(
        _sc_body,
        out_type=jax.ShapeDtypeStruct((B * L, D), jnp.float32),
        mesh=plsc.VectorSubcoreMesh(core_axis_name="c", subcore_axis_name="s"),
        compiler_params=pltpu.CompilerParams(needs_layout_passes=False),
        scratch_types=[
            pltpu.VMEM((L,), jnp.int32),
            pltpu.VMEM((NCH, CHUNK), jnp.int32),
            pltpu.VMEM((NBUF, CHUNK, D), jnp.float32),
            pltpu.SemaphoreType.DMA((NBUF,)),
            pltpu.SemaphoreType.DMA((NBUF,)),
        ],
    )(x2, at)
    return run.reshape(B, L, D)


# speedup vs baseline: 2.1620x; 1.0009x over previous
import jax
import jax.numpy as jnp
from jax import lax
from jax.experimental import pallas as pl
from jax.experimental.pallas import tpu as pltpu
from jax.experimental.pallas import tpu_sc as plsc

B, L, D = 16, 4096, 128
NC, NS = 2, 16
HALF = L // 2
LANES = 16
NG = HALF // LANES
CHUNK = 128
NCH = HALF // CHUNK
GPC = CHUNK // LANES
NBUF = 4


def _sc_body(x_hbm, atom_hbm, out_hbm, atom_v, idx_v, xbuf, gsems, ssems):
    c = lax.axis_index("c")
    s = lax.axis_index("s")
    wid = c * NS + s
    b = wid // 2
    h = wid % 2
    row_base = b * L
    my_base = row_base + h * HALF

    gat = [None] * NBUF
    scat = [None] * NBUF
    for j in range(2):
        gat[j] = pltpu.async_copy(
            x_hbm.at[pl.ds(my_base + j * CHUNK, CHUNK)], xbuf.at[j], gsems.at[j])

    pltpu.sync_copy(atom_hbm.at[pl.ds(row_base, L)], atom_v)

    def count_half(half):
        def grp(g, acc):
            accs = []
            for u in range(4):
                a = atom_v[pl.ds(half * HALF + (4 * g + u) * LANES, LANES)]
                accs.append(jnp.where(a < 3, 1, 0).astype(jnp.int32))
            return acc + ((accs[0] + accs[1]) + (accs[2] + accs[3]))
        acc = lax.fori_loop(0, NG // 4, grp, jnp.zeros((LANES,), jnp.int32))
        return jnp.sum(acc)

    c0 = count_half(0)
    c1 = count_half(1)

    bb_base = row_base + h * c0
    ot_base = row_base + (c0 + c1) + h * (HALF - c0)
    lane = lax.iota(jnp.int32, LANES)

    run = jnp.int32(0)
    for j in range(NCH):
        flags, css = [], []
        for g in range(GPC):
            a = atom_v[pl.ds(h * HALF + j * CHUNK + g * LANES, LANES)]
            flag = a < 3
            flags.append(flag)
            css.append(plsc.cumsum(jnp.where(flag, 1, 0).astype(jnp.int32)))
        pre = run
        for g in range(GPC):
            cs = css[g]
            p = lane + (j * CHUNK + g * LANES)
            ib = pre + cs
            dest = jnp.where(flags[g], bb_base + ib - 1, ot_base + p - ib)
            idx_v[j, pl.ds(g * LANES, LANES)] = dest
            pre = pre + cs[15]
        run = pre

        jn = j + 2
        if jn < NCH:
            bn = jn % NBUF
            if scat[bn] is not None:
                scat[bn].wait()
            gat[bn] = pltpu.async_copy(
                x_hbm.at[pl.ds(my_base + jn * CHUNK, CHUNK)], xbuf.at[bn],
                gsems.at[bn])
        bj = j % NBUF
        gat[bj].wait()
        scat[bj] = pltpu.async_copy(xbuf.at[bj], out_hbm.at[idx_v.at[j]],
                                    ssems.at[bj])

    for bj in range(NBUF):
        scat[bj].wait()


@jax.jit
def kernel(x, atom_type, aa_type):
    del aa_type
    x2 = x.reshape(B * L, D)
    at = atom_type.astype(jnp.int32).reshape(B * L)
    run = pl.kernel(
        _sc_body,
        out_type=jax.ShapeDtypeStruct((B * L, D), jnp.float32),
        mesh=plsc.VectorSubcoreMesh(core_axis_name="c", subcore_axis_name="s"),
        compiler_params=pltpu.CompilerParams(needs_layout_passes=False),
        scratch_types=[
            pltpu.VMEM((L,), jnp.int32),
            pltpu.VMEM((NCH, CHUNK), jnp.int32),
            pltpu.VMEM((NBUF, CHUNK, D), jnp.float32),
            pltpu.SemaphoreType.DMA((NBUF,)),
            pltpu.SemaphoreType.DMA((NBUF,)),
        ],
    )(x2, at)
    return run.reshape(B, L, D)


# speedup vs baseline: 2.1727x; 1.0049x over previous
import jax
import jax.numpy as jnp
from jax import lax
from jax.experimental import pallas as pl
from jax.experimental.pallas import tpu as pltpu
from jax.experimental.pallas import tpu_sc as plsc

B, L, D = 16, 4096, 128
NC, NS = 2, 16
HALF = L // 2
LANES = 16
NG = HALF // LANES
CHUNK = 128
NCH = HALF // CHUNK
GPC = CHUNK // LANES
NBUF = 4


def _sc_body(x_hbm, atom_hbm, out_hbm, atom_v, idx_v, xbuf, gsems, ssems):
    c = lax.axis_index("c")
    s = lax.axis_index("s")
    wid = c * NS + s
    b = wid // 2
    h = wid % 2
    row_base = b * L
    my_base = row_base + h * HALF

    gat = [None] * NBUF
    scat = [None] * NBUF
    for j in range(2):
        gat[j] = pltpu.async_copy(
            x_hbm.at[pl.ds(my_base + j * CHUNK, CHUNK)], xbuf.at[j], gsems.at[j])

    pltpu.sync_copy(atom_hbm.at[b], atom_v)

    def count_half(half):
        def grp(g, acc):
            accs = []
            for u in range(4):
                a = atom_v[pl.ds(half * HALF + (4 * g + u) * LANES, LANES)]
                accs.append(jnp.where(a < 3, 1, 0).astype(jnp.int32))
            return acc + ((accs[0] + accs[1]) + (accs[2] + accs[3]))
        acc = lax.fori_loop(0, NG // 4, grp, jnp.zeros((LANES,), jnp.int32))
        return jnp.sum(acc)

    c0 = count_half(0)
    c1 = count_half(1)

    bb_base = row_base + h * c0
    ot_base = row_base + (c0 + c1) + h * (HALF - c0)
    lane = lax.iota(jnp.int32, LANES)

    run = jnp.int32(0)
    for j in range(NCH):
        def grp(g, run):
            a = atom_v[pl.ds(h * HALF + j * CHUNK + g * LANES, LANES)]
            flag = a < 3
            f = jnp.where(flag, 1, 0).astype(jnp.int32)
            cs = plsc.cumsum(f)
            p = lane + (j * CHUNK + g * LANES)
            ib = run + cs
            dest = jnp.where(flag, bb_base + ib - 1, ot_base + p - ib)
            idx_v[j, pl.ds(g * LANES, LANES)] = dest
            return run + jnp.sum(f)
        run = lax.fori_loop(0, GPC, grp, run)

        jn = j + 2
        if jn < NCH:
            bn = jn % NBUF
            if scat[bn] is not None:
                scat[bn].wait()
            gat[bn] = pltpu.async_copy(
                x_hbm.at[pl.ds(my_base + jn * CHUNK, CHUNK)], xbuf.at[bn],
                gsems.at[bn])
        bj = j % NBUF
        gat[bj].wait()
        scat[bj] = pltpu.async_copy(xbuf.at[bj], out_hbm.at[idx_v.at[j]],
                                    ssems.at[bj])

    for bj in range(NBUF):
        scat[bj].wait()


@jax.jit
def kernel(x, atom_type, aa_type):
    del aa_type
    x2 = x.reshape(B * L, D)
    at = atom_type.astype(jnp.int32)
    run = pl.kernel(
        _sc_body,
        out_type=jax.ShapeDtypeStruct((B * L, D), jnp.float32),
        mesh=plsc.VectorSubcoreMesh(core_axis_name="c", subcore_axis_name="s"),
        compiler_params=pltpu.CompilerParams(needs_layout_passes=False),
        scratch_types=[
            pltpu.VMEM((L,), jnp.int32),
            pltpu.VMEM((NCH, CHUNK), jnp.int32),
            pltpu.VMEM((NBUF, CHUNK, D), jnp.float32),
            pltpu.SemaphoreType.DMA((NBUF,)),
            pltpu.SemaphoreType.DMA((NBUF,)),
        ],
    )(x2, at)
    return run.reshape(B, L, D)
